# Initial kernel scaffold; baseline (speedup 1.0000x reference)
import jax
import jax.numpy as jnp
from jax.experimental import pallas as pl


def kernel(x, W_enc, b_enc, W_dec, b_dec):
    raise NotImplementedError("write your pallas kernel here")


# speedup vs baseline: 7.1489x; 7.1489x over previous
import functools

import jax
import jax.numpy as jnp
from jax import lax
from jax.experimental import pallas as pl
from jax.experimental.pallas import tpu as pltpu

BT = 128
D_IN = 768
D_SAE = 24576
TOPK = 64
F_BLK = 2048
NB = D_SAE // F_BLK


def _body(x_ref, we_ref, be_ref, wd_ref, bd_ref,
          recon_ref, sparse_ref, acts_ref, t_ref, j_ref):
    step = pl.program_id(0)

    @pl.when(step < NB)
    def _encode():
        xc = x_ref[...] - bd_ref[...]
        pre = lax.dot_general(xc, we_ref[...], (((1,), (0,)), ((), ())),
                              preferred_element_type=jnp.float32)
        pre = pre + be_ref[...]
        off = pl.multiple_of(step * F_BLK, F_BLK)
        acts_ref[:, pl.ds(off, F_BLK)] = jnp.maximum(pre, 0.0)

    @pl.when(step == NB)
    def _select():
        def count_ge(c):
            def chunk(i, acc):
                off = pl.multiple_of(i * F_BLK, F_BLK)
                bits = lax.bitcast_convert_type(
                    acts_ref[:, pl.ds(off, F_BLK)], jnp.int32)
                return acc + jnp.sum((bits >= c).astype(jnp.int32),
                                     axis=1, keepdims=True)
            return lax.fori_loop(0, NB, chunk,
                                 jnp.zeros((BT, 1), jnp.int32))

        def bit_step(i, t):
            c_test = t | jnp.left_shift(jnp.int32(1), 30 - i)
            cnt = count_ge(c_test)
            return jnp.where(cnt >= TOPK, c_test, t)
        t = lax.fori_loop(0, 31, bit_step, jnp.zeros((BT, 1), jnp.int32))

        n_gt = count_ge(t + 1)
        n_need = TOPK - n_gt

        def count_eq_below(c):
            def chunk(i, acc):
                off = pl.multiple_of(i * F_BLK, F_BLK)
                bits = lax.bitcast_convert_type(
                    acts_ref[:, pl.ds(off, F_BLK)], jnp.int32)
                idx = lax.broadcasted_iota(jnp.int32, (BT, F_BLK), 1) + off
                hit = (bits == t) & (idx < c)
                return acc + jnp.sum(hit.astype(jnp.int32),
                                     axis=1, keepdims=True)
            return lax.fori_loop(0, NB, chunk,
                                 jnp.zeros((BT, 1), jnp.int32))

        def idx_step(i, jcur):
            c_test = jcur | jnp.left_shift(jnp.int32(1), 14 - i)
            cnt = count_eq_below(c_test)
            return jnp.where(cnt < n_need, c_test, jcur)
        j = lax.fori_loop(0, 15, idx_step, jnp.zeros((BT, 1), jnp.int32))

        t_ref[...] = t
        j_ref[...] = j

    @pl.when(step >= NB)
    def _decode():
        blk = step - NB
        off = pl.multiple_of(blk * F_BLK, F_BLK)
        a = acts_ref[:, pl.ds(off, F_BLK)]
        bits = lax.bitcast_convert_type(a, jnp.int32)
        idx = lax.broadcasted_iota(jnp.int32, (BT, F_BLK), 1) + off
        t = t_ref[...]
        j = j_ref[...]
        keep = (bits > t) | ((bits == t) & (idx <= j))
        sp = jnp.where(keep, a, 0.0)
        sparse_ref[...] = sp

        part = lax.dot_general(sp, wd_ref[...], (((1,), (0,)), ((), ())),
                               preferred_element_type=jnp.float32)

        @pl.when(step == NB)
        def _init():
            recon_ref[...] = bd_ref[...] + jnp.zeros((BT, D_IN), jnp.float32)

        recon_ref[...] += part


@functools.partial(jax.jit, static_argnames=("interpret",))
def _run(x2d, w_enc, b_enc2d, w_dec, b_dec2d, interpret=False):
    grid = (2 * NB,)
    recon, sparse = pl.pallas_call(
        _body,
        grid=grid,
        in_specs=[
            pl.BlockSpec((BT, D_IN), lambda i: (0, 0)),
            pl.BlockSpec((D_IN, F_BLK), lambda i: (0, jnp.minimum(i, NB - 1))),
            pl.BlockSpec((1, F_BLK), lambda i: (0, jnp.minimum(i, NB - 1))),
            pl.BlockSpec((F_BLK, D_IN), lambda i: (jnp.maximum(i - NB, 0), 0)),
            pl.BlockSpec((1, D_IN), lambda i: (0, 0)),
        ],
        out_specs=[
            pl.BlockSpec((BT, D_IN), lambda i: (0, 0)),
            pl.BlockSpec((BT, F_BLK), lambda i: (0, jnp.maximum(i - NB, 0))),
        ],
        out_shape=[
            jax.ShapeDtypeStruct((BT, D_IN), jnp.float32),
            jax.ShapeDtypeStruct((BT, D_SAE), jnp.float32),
        ],
        scratch_shapes=[
            pltpu.VMEM((BT, D_SAE), jnp.float32),
            pltpu.VMEM((BT, 1), jnp.int32),
            pltpu.VMEM((BT, 1), jnp.int32),
        ],
        interpret=interpret,
    )(x2d, w_enc, b_enc2d, w_dec, b_dec2d)
    return recon, sparse


def kernel(x, W_enc, b_enc, W_dec, b_dec):
    b, s, d_in = x.shape
    x2d = x.reshape(b * s, d_in)
    recon, sparse = _run(x2d, W_enc, b_enc.reshape(1, -1),
                         W_dec, b_dec.reshape(1, -1))
    return recon.reshape(b, s, d_in), sparse.reshape(b, s, -1)
